# baseline (speedup 1.0000x reference)
import functools

import jax
import jax.numpy as jnp
from jax import lax
from jax.experimental import pallas as pl
from jax.experimental.pallas import tpu as pltpu
from jax.experimental.pallas import tpu_sc as plsc

NU = 10000
NI = 10000
D = 128
LAM = 0.5
GB = 3.5
E_RATE = 320000
E_PRED = 100000

NC = 2
NS = 16
NW = NC * NS
L = 16

NI_PAD = 10240
E_TILE = E_RATE // NW
EK = 80
NCH_RATE = E_TILE // EK
RK = 80
NCH_ROWS = NU // RK
BIGC = 625
EP_PAD = 102400
EP_TILE = EP_PAD // NW
EK3 = 128
NCH_PRED = EP_TILE // EK3
QK = 32
NQ = EP_TILE // QK
NB3 = 6

_MESH = plsc.VectorSubcoreMesh(
    core_axis_name="c", subcore_axis_name="s", num_cores=NC, num_subcores=NS)


def _dyn_gather(x, idx):
  dnums = lax.GatherDimensionNumbers(
      offset_dims=(), collapsed_slice_dims=(0,), start_index_map=(0,))
  return lax.gather(x, idx[:, None], dnums, slice_sizes=(1,),
                    mode=lax.GatherScatterMode.PROMISE_IN_BOUNDS)


def _zero_fill(ref, n):
  zeros16 = jnp.zeros((L,), jnp.float32)

  def body(i, _):
    ref[pl.ds(i * L, L)] = zeros16
    return 0

  lax.fori_loop(0, n, body, 0)


@functools.partial(
    pl.kernel,
    out_type=(
        jax.ShapeDtypeStruct((NC * NI_PAD,), jnp.float32),
        jax.ShapeDtypeStruct((NW * 8 * L,), jnp.float32),
    ),
    mesh=_MESH,
    scratch_types=[
        pltpu.VMEM((NCH_RATE, EK), jnp.int32),
        pltpu.VMEM((EK,), jnp.float32),
        pltpu.VMEM((2048,), jnp.float32),
        pltpu.VMEM((8 * L,), jnp.float32),
        pltpu.VMEM_SHARED((NI_PAD,), jnp.float32),
        pltpu.SemaphoreType.DMA,
    ],
)
def _k1_hist_reg(rate_hbm, pqu_hbm, pqi_hbm, ywi_hbm, bu_hbm, bi_hbm,
                 hist_out, reg_out, ivd, ones_v, vbuf, pbuf, hist_sh, hsem):
  c = lax.axis_index("c")
  s = lax.axis_index("s")
  w = c * NS + s
  zeros16 = jnp.zeros((L,), jnp.float32)
  ones16 = jnp.ones((L,), jnp.float32)

  for g in range(EK // L):
    ones_v[pl.ds(g * L, L)] = ones16
  _zero_fill(vbuf, 2048 // L)
  pltpu.sync_copy(vbuf.at[pl.ds(0, NI_PAD // NS)],
                  hist_sh.at[pl.ds(s * (NI_PAD // NS), NI_PAD // NS)])
  pltpu.sync_copy(rate_hbm.at[1, w], ivd)
  plsc.subcore_barrier()

  def fire(j, _):
    pltpu.async_copy(ones_v, hist_sh.at[ivd.at[j]], hsem, add=True)
    return 0

  def drain(j, _):
    pltpu.make_async_copy(ones_v, hist_sh.at[ivd.at[j]], hsem).wait()
    return 0

  def grp(o, _):
    lax.fori_loop(o * 25, o * 25 + 25, fire, 0)
    lax.fori_loop(o * 25, o * 25 + 25, drain, 0)
    return 0

  lax.fori_loop(0, NCH_RATE // 25, grp, 0)
  plsc.subcore_barrier()
  pltpu.sync_copy(hist_sh.at[pl.ds(s * (NI_PAD // NS), NI_PAD // NS)],
                  hist_out.at[pl.ds(c * NI_PAD + s * (NI_PAD // NS),
                                    NI_PAD // NS)])

  _zero_fill(pbuf, 8)

  def ssq_big(arr_hbm, k):
    def chunk_body(i, _):
      cidx = w + NW * i

      @pl.when(cidx < BIGC)
      def _():
        pltpu.sync_copy(arr_hbm.at[pl.ds(cidx * 2048, 2048)], vbuf)

        def red(f, aa):
          v = vbuf[pl.ds(f * L, L)]
          return aa + v * v

        acc = lax.fori_loop(0, 2048 // L, red, zeros16)
        pbuf[pl.ds(k * L, L)] = pbuf[pl.ds(k * L, L)] + acc

      return 0

    lax.fori_loop(0, 20, chunk_body, 0)

  def ssq_small(arr_hbm, k):
    @pl.when(w < NU // 400)
    def _():
      pltpu.sync_copy(arr_hbm.at[pl.ds(w * 400, 400)], vbuf.at[pl.ds(0, 400)])

      def red(f, aa):
        v = vbuf[pl.ds(f * L, L)]
        return aa + v * v

      acc = lax.fori_loop(0, 400 // L, red, zeros16)
      pbuf[pl.ds(k * L, L)] = acc

  ssq_small(bu_hbm, 0)
  ssq_big(pqu_hbm, 1)
  ssq_small(bi_hbm, 2)
  ssq_big(pqi_hbm, 3)
  ssq_big(ywi_hbm, 4)
  pltpu.sync_copy(pbuf, reg_out.at[pl.ds(w * 8 * L, 8 * L)])


@functools.partial(
    pl.kernel,
    out_type=jax.ShapeDtypeStruct((NI, D), jnp.float32),
    mesh=_MESH,
    scratch_types=[
        pltpu.VMEM((RK,), jnp.float32),
        pltpu.VMEM((RK,), jnp.float32),
        pltpu.VMEM((RK, D), jnp.float32),
    ],
)
def _k2a_scale(ywi_hbm, hist_hbm, syw_out, h0v, h1v, rows_v):
  c = lax.axis_index("c")
  s = lax.axis_index("s")
  w = c * NS + s
  iot = lax.iota(jnp.int32, L)

  def chunk(i, _):
    cidx = w + NW * i

    @pl.when(cidx < NCH_ROWS)
    def _():
      r0 = cidx * RK
      pltpu.sync_copy(hist_hbm.at[pl.ds(r0, RK)], h0v)
      pltpu.sync_copy(hist_hbm.at[pl.ds(NI_PAD + r0, RK)], h1v)
      pltpu.sync_copy(ywi_hbm.at[pl.ds(r0, RK)], rows_v)

      def grp(g, _2):
        h = h0v[pl.ds(g * L, L)] + h1v[pl.ds(g * L, L)]
        n = 1.0 / jnp.maximum(h, 1.0)

        def rloop(r2, _3):
          nr = _dyn_gather(n, jnp.broadcast_to(r2, (L,)))
          r = g * L + r2
          for f in range(D // L):
            sl = pl.ds(f * L, L)
            rows_v[r, sl] = rows_v[r, sl] * nr
          return 0

        lax.fori_loop(0, L, rloop, 0)
        return 0

      lax.fori_loop(0, RK // L, grp, 0)
      pltpu.sync_copy(rows_v, syw_out.at[pl.ds(r0, RK)])

    return 0

  lax.fori_loop(0, (NCH_ROWS + NW - 1) // NW, chunk, 0)


@functools.partial(
    pl.kernel,
    out_type=jax.ShapeDtypeStruct((NC, NU, D), jnp.float32),
    mesh=_MESH,
    scratch_types=[
        pltpu.VMEM((3, EK), jnp.int32),
        pltpu.VMEM((2, EK), jnp.int32),
        pltpu.VMEM((2, EK, D), jnp.float32),
        pltpu.VMEM((16, D), jnp.float32),
        pltpu.VMEM_SHARED((NU, D), jnp.float32),
        pltpu.SemaphoreType.DMA((3,)),
        pltpu.SemaphoreType.DMA((2,)),
        pltpu.SemaphoreType.DMA((2,)),
        pltpu.SemaphoreType.DMA((2,)),
    ],
)
def _k2b_segsum(rate_hbm, syw_hbm, acc_out, ivs, ivd, rows4, zbuf, acc_sh,
                issem, idsem, gsem, ssem):
  c = lax.axis_index("c")
  s = lax.axis_index("s")
  w = c * NS + s
  zeros16 = jnp.zeros((L,), jnp.float32)

  for r in range(16):
    for f in range(D // L):
      zbuf[r, pl.ds(f * L, L)] = zeros16

  n_grp = NU // 16

  def zloop(i, _):
    g16 = s + NS * i

    @pl.when(g16 < n_grp)
    def _():
      pltpu.sync_copy(zbuf, acc_sh.at[pl.ds(g16 * 16, 16)])

    return 0

  lax.fori_loop(0, (n_grp + NS - 1) // NS, zloop, 0)
  plsc.subcore_barrier()

  base_s = w * E_TILE
  base_d = E_RATE + w * E_TILE

  def isrc_start(j, k):
    pltpu.async_copy(rate_hbm.at[pl.ds(base_s + j * EK, EK)], ivs.at[k],
                     issem.at[k])

  def isrc_wait(j, k):
    pltpu.make_async_copy(rate_hbm.at[pl.ds(base_s + j * EK, EK)], ivs.at[k],
                          issem.at[k]).wait()

  def idst_start(j, k):
    pltpu.async_copy(rate_hbm.at[pl.ds(base_d + j * EK, EK)], ivd.at[k],
                     idsem.at[k])

  def idst_wait(j, k):
    pltpu.make_async_copy(rate_hbm.at[pl.ds(base_d + j * EK, EK)], ivd.at[k],
                          idsem.at[k]).wait()

  def gather_start(j, b):
    pltpu.async_copy(syw_hbm.at[ivd.at[lax.rem(j, 2)]], rows4.at[b],
                     gsem.at[b])

  def gather_wait(j, b):
    pltpu.make_async_copy(syw_hbm.at[ivd.at[lax.rem(j, 2)]], rows4.at[b],
                          gsem.at[b]).wait()

  def scat_start(j, b):
    pltpu.async_copy(rows4.at[b], acc_sh.at[ivs.at[lax.rem(j, 3)]],
                     ssem.at[b], add=True)

  def scat_wait(j, b):
    pltpu.make_async_copy(rows4.at[b], acc_sh.at[ivs.at[lax.rem(j, 3)]],
                          ssem.at[b]).wait()

  idst_start(0, 0)
  idst_start(1, 1)
  isrc_start(0, 0)
  isrc_start(1, 1)
  isrc_start(2, 2)
  idst_wait(0, 0)
  gather_start(0, 0)

  def eloop(j, _):
    b = lax.rem(j, 2)
    gather_wait(j, b)
    isrc_wait(j, lax.rem(j, 3))
    scat_start(j, b)

    @pl.when(j >= 1)
    def _():
      scat_wait(j - 1, 1 - b)

    @pl.when(j + 2 < NCH_RATE)
    def _():
      isrc_start(j + 2, lax.rem(j + 2, 3))

    @pl.when(j + 1 < NCH_RATE)
    def _():
      idst_wait(j + 1, 1 - b)
      gather_start(j + 1, 1 - b)

    @pl.when(j + 2 < NCH_RATE)
    def _():
      idst_start(j + 2, b)

    return 0

  lax.fori_loop(0, NCH_RATE, eloop, 0)
  scat_wait(NCH_RATE - 1, lax.rem(NCH_RATE - 1, 2))
  plsc.subcore_barrier()

  def wloop(i, _):
    g16 = s + NS * i

    @pl.when(g16 < n_grp)
    def _():
      r0 = g16 * 16
      pltpu.sync_copy(acc_sh.at[pl.ds(r0, 16)], acc_out.at[c, pl.ds(r0, 16)])

    return 0

  lax.fori_loop(0, (n_grp + NS - 1) // NS, wloop, 0)


@functools.partial(
    pl.kernel,
    out_type=jax.ShapeDtypeStruct((NU, D), jnp.float32),
    mesh=_MESH,
    scratch_types=[
        pltpu.VMEM((RK, D), jnp.float32),
        pltpu.VMEM((RK, D), jnp.float32),
        pltpu.VMEM((RK, D), jnp.float32),
        pltpu.VMEM((RK, D), jnp.float32),
    ],
)
def _k2c_hu(acc_hbm, pqu_hbm, hu_out, a0v, a1v, pv, hv):
  c = lax.axis_index("c")
  s = lax.axis_index("s")
  w = c * NS + s

  def chunk(i, _):
    cidx = w + NW * i

    @pl.when(cidx < NCH_ROWS)
    def _():
      r0 = cidx * RK
      pltpu.sync_copy(acc_hbm.at[0, pl.ds(r0, RK)], a0v)
      pltpu.sync_copy(acc_hbm.at[1, pl.ds(r0, RK)], a1v)
      pltpu.sync_copy(pqu_hbm.at[pl.ds(r0, RK)], pv)

      def rloop(r, _2):
        for f in range(D // L):
          sl = pl.ds(f * L, L)
          hv[r, sl] = a0v[r, sl] + a1v[r, sl] + pv[r, sl]
        return 0

      lax.fori_loop(0, RK, rloop, 0)
      pltpu.sync_copy(hv, hu_out.at[pl.ds(r0, RK)])

    return 0

  lax.fori_loop(0, (NCH_ROWS + NW - 1) // NW, chunk, 0)


@functools.partial(
    pl.kernel,
    out_type=(
        jax.ShapeDtypeStruct((EP_PAD,), jnp.float32),
        jax.ShapeDtypeStruct((EP_PAD,), jnp.float32),
    ),
    mesh=_MESH,
    scratch_types=[
        pltpu.VMEM((NCH_PRED, EK3), jnp.int32),
        pltpu.VMEM((NCH_PRED, EK3), jnp.int32),
        pltpu.VMEM((NB3, QK, D), jnp.float32),
        pltpu.VMEM((NB3, QK, D), jnp.float32),
        pltpu.VMEM((NB3, QK), jnp.float32),
        pltpu.VMEM((NB3, QK), jnp.float32),
        pltpu.VMEM((NB3, QK), jnp.float32),
        pltpu.SemaphoreType.DMA((NB3,)),
        pltpu.SemaphoreType.DMA((NB3,)),
        pltpu.SemaphoreType.DMA((NB3,)),
        pltpu.SemaphoreType.DMA((NB3,)),
        pltpu.SemaphoreType.DMA((NB3,)),
    ],
)
def _k3_score(pos_hbm, neg_hbm, hu_hbm, pqi_hbm, bu_hbm, bi_hbm,
              pos_out, neg_out, ivs, ivd, uv2, pv2, bu2, bi2, sv2,
              usem, psem, busem, bisem, osem):
  c = lax.axis_index("c")
  s = lax.axis_index("s")
  w = c * NS + s
  iot = lax.iota(jnp.int32, L)
  qpr = EK3 // QK

  for edges_hbm, out_hbm in ((pos_hbm, pos_out), (neg_hbm, neg_out)):

    pltpu.sync_copy(edges_hbm.at[0, w], ivs)
    pltpu.sync_copy(edges_hbm.at[1, w], ivd)

    def sidx(j):
      return ivs.at[lax.div(j, qpr), pl.ds(lax.rem(j, qpr) * QK, QK)]

    def didx(j):
      return ivd.at[lax.div(j, qpr), pl.ds(lax.rem(j, qpr) * QK, QK)]

    def gathers_start(j, b):
      pltpu.async_copy(hu_hbm.at[sidx(j)], uv2.at[b], usem.at[b])
      pltpu.async_copy(pqi_hbm.at[didx(j)], pv2.at[b], psem.at[b])
      pltpu.async_copy(bu_hbm.at[sidx(j)], bu2.at[b], busem.at[b])
      pltpu.async_copy(bi_hbm.at[didx(j)], bi2.at[b], bisem.at[b])

    def gathers_wait(j, b):
      pltpu.make_async_copy(hu_hbm.at[sidx(j)], uv2.at[b],
                            usem.at[b]).wait()
      pltpu.make_async_copy(pqi_hbm.at[didx(j)], pv2.at[b],
                            psem.at[b]).wait()
      pltpu.make_async_copy(bu_hbm.at[sidx(j)], bu2.at[b],
                            busem.at[b]).wait()
      pltpu.make_async_copy(bi_hbm.at[didx(j)], bi2.at[b],
                            bisem.at[b]).wait()

    def out_start(j, b, out_hbm=out_hbm):
      pltpu.async_copy(sv2.at[b], out_hbm.at[pl.ds(w * EP_TILE + j * QK,
                                                   QK)], osem.at[b])

    def out_wait(j, b, out_hbm=out_hbm):
      pltpu.make_async_copy(sv2.at[b], out_hbm.at[pl.ds(w * EP_TILE + j * QK,
                                                        QK)],
                            osem.at[b]).wait()

    for b in range(NB3 - 1):
      gathers_start(b, b)

    def chunk(j, _):
      b = lax.rem(j, NB3)
      gathers_wait(j, b)

      @pl.when(j + NB3 - 1 < NQ)
      def _():
        gathers_start(j + NB3 - 1, lax.rem(j + NB3 - 1, NB3))

      @pl.when(j >= NB3)
      def _():
        out_wait(j - NB3, b)

      for g in range(QK // L):

        def edge_body(r2, merged, g=g, b=b):
          r = g * L + r2
          acc = jnp.zeros((L,), jnp.float32)
          for f in range(D // L):
            sl = pl.ds(f * L, L)
            acc = acc + uv2[b, r, sl] * pv2[b, r, sl]
          for sh in (8, 4, 2, 1):
            acc = acc + _dyn_gather(acc, jnp.bitwise_xor(iot, sh))
          return jnp.where(iot == r2, acc, merged)

        dots = lax.fori_loop(0, L, edge_body, jnp.zeros((L,), jnp.float32))
        sc = (dots + bu2[b, pl.ds(g * L, L)] + bi2[b, pl.ds(g * L, L)] + GB)
        sv2[b, pl.ds(g * L, L)] = sc
      out_start(j, b)
      return 0

    lax.fori_loop(0, NQ, chunk, 0)
    for dj in range(NB3):
      out_wait(NQ - NB3 + dj, lax.rem(NQ - NB3 + dj, NB3))


def kernel(rate_edges, trust_edges, pos_edges, neg_edges,
           pq_user, pq_item, yw_user, yw_item, b_user, b_item):
  del trust_edges, yw_user
  pqu_flat = pq_user.reshape(-1)
  pqi_flat = pq_item.reshape(-1)
  ywi_flat = yw_item.reshape(-1)
  bu_flat = b_user.reshape(-1)
  bi_flat = b_item.reshape(-1)

  rate4 = rate_edges.reshape(2, NW, NCH_RATE, EK)
  hist, regp = _k1_hist_reg(rate4, pqu_flat, pqi_flat, ywi_flat,
                            bu_flat, bi_flat)
  syw = _k2a_scale(yw_item, hist)
  acc = _k2b_segsum(rate_edges.reshape(-1), syw)
  h_u = _k2c_hu(acc, pq_user)

  pos4 = jnp.pad(pos_edges, ((0, 0), (0, EP_PAD - E_PRED))).reshape(
      2, NW, NCH_PRED, EK3)
  neg4 = jnp.pad(neg_edges, ((0, 0), (0, EP_PAD - E_PRED))).reshape(
      2, NW, NCH_PRED, EK3)
  pos_s, neg_s = _k3_score(pos4, neg4, h_u, pq_item, bu_flat, bi_flat)
  pos_score = pos_s[:E_PRED, None]
  neg_score = neg_s[:E_PRED, None]

  ssq = regp.reshape(NW, 8, L)[:, :5, :].sum(axis=(0, 2))
  nb_u, np_u, nb_i, np_i, ny_i = [jnp.sqrt(ssq[k]) for k in range(5)]
  reg_loss = ((LAM * E_RATE / NU) * (nb_u + np_u) +
              (LAM * E_RATE / NI) * (nb_i + np_i + ny_i))
  return (pos_score, neg_score, reg_loss)


# speedup vs baseline: 1.0185x; 1.0185x over previous
import functools

import jax
import jax.numpy as jnp
from jax import lax
from jax.experimental import pallas as pl
from jax.experimental.pallas import tpu as pltpu
from jax.experimental.pallas import tpu_sc as plsc

NU = 10000
NI = 10000
D = 128
LAM = 0.5
GB = 3.5
E_RATE = 320000
E_PRED = 100000

NC = 2
NS = 16
NW = NC * NS
L = 16

NI_PAD = 10240
E_TILE = E_RATE // NW
EK = 80
NCH_RATE = E_TILE // EK
RK = 80
NCH_ROWS = NU // RK
BIGC = 625
EP_PAD = 102400
EP_TILE = EP_PAD // NW
EK3 = 128
NCH_PRED = EP_TILE // EK3
QK = 64
NQ = EP_TILE // QK
NB3 = 6

_MESH = plsc.VectorSubcoreMesh(
    core_axis_name="c", subcore_axis_name="s", num_cores=NC, num_subcores=NS)


def _dyn_gather(x, idx):
  dnums = lax.GatherDimensionNumbers(
      offset_dims=(), collapsed_slice_dims=(0,), start_index_map=(0,))
  return lax.gather(x, idx[:, None], dnums, slice_sizes=(1,),
                    mode=lax.GatherScatterMode.PROMISE_IN_BOUNDS)


def _zero_fill(ref, n):
  zeros16 = jnp.zeros((L,), jnp.float32)

  def body(i, _):
    ref[pl.ds(i * L, L)] = zeros16
    return 0

  lax.fori_loop(0, n, body, 0)


@functools.partial(
    pl.kernel,
    out_type=(
        jax.ShapeDtypeStruct((NC * NI_PAD,), jnp.float32),
        jax.ShapeDtypeStruct((NW * 8 * L,), jnp.float32),
    ),
    mesh=_MESH,
    scratch_types=[
        pltpu.VMEM((NCH_RATE, EK), jnp.int32),
        pltpu.VMEM((EK,), jnp.float32),
        pltpu.VMEM((2048,), jnp.float32),
        pltpu.VMEM((8 * L,), jnp.float32),
        pltpu.VMEM_SHARED((NI_PAD,), jnp.float32),
        pltpu.SemaphoreType.DMA,
    ],
)
def _k1_hist_reg(rate_hbm, pqu_hbm, pqi_hbm, ywi_hbm, bu_hbm, bi_hbm,
                 hist_out, reg_out, ivd, ones_v, vbuf, pbuf, hist_sh, hsem):
  c = lax.axis_index("c")
  s = lax.axis_index("s")
  w = c * NS + s
  zeros16 = jnp.zeros((L,), jnp.float32)
  ones16 = jnp.ones((L,), jnp.float32)

  for g in range(EK // L):
    ones_v[pl.ds(g * L, L)] = ones16
  _zero_fill(vbuf, 2048 // L)
  pltpu.sync_copy(vbuf.at[pl.ds(0, NI_PAD // NS)],
                  hist_sh.at[pl.ds(s * (NI_PAD // NS), NI_PAD // NS)])
  pltpu.sync_copy(rate_hbm.at[1, w], ivd)
  plsc.subcore_barrier()

  def fire(j, _):
    pltpu.async_copy(ones_v, hist_sh.at[ivd.at[j]], hsem, add=True)
    return 0

  def drain(j, _):
    pltpu.make_async_copy(ones_v, hist_sh.at[ivd.at[j]], hsem).wait()
    return 0

  def grp(o, _):
    lax.fori_loop(o * 25, o * 25 + 25, fire, 0)
    lax.fori_loop(o * 25, o * 25 + 25, drain, 0)
    return 0

  lax.fori_loop(0, NCH_RATE // 25, grp, 0)
  plsc.subcore_barrier()
  pltpu.sync_copy(hist_sh.at[pl.ds(s * (NI_PAD // NS), NI_PAD // NS)],
                  hist_out.at[pl.ds(c * NI_PAD + s * (NI_PAD // NS),
                                    NI_PAD // NS)])

  _zero_fill(pbuf, 8)

  def ssq_big(arr_hbm, k):
    def chunk_body(i, _):
      cidx = w + NW * i

      @pl.when(cidx < BIGC)
      def _():
        pltpu.sync_copy(arr_hbm.at[pl.ds(cidx * 2048, 2048)], vbuf)

        def red(f, aa):
          v = vbuf[pl.ds(f * L, L)]
          return aa + v * v

        acc = lax.fori_loop(0, 2048 // L, red, zeros16)
        pbuf[pl.ds(k * L, L)] = pbuf[pl.ds(k * L, L)] + acc

      return 0

    lax.fori_loop(0, 20, chunk_body, 0)

  def ssq_small(arr_hbm, k):
    @pl.when(w < NU // 400)
    def _():
      pltpu.sync_copy(arr_hbm.at[pl.ds(w * 400, 400)], vbuf.at[pl.ds(0, 400)])

      def red(f, aa):
        v = vbuf[pl.ds(f * L, L)]
        return aa + v * v

      acc = lax.fori_loop(0, 400 // L, red, zeros16)
      pbuf[pl.ds(k * L, L)] = acc

  ssq_small(bu_hbm, 0)
  ssq_big(pqu_hbm, 1)
  ssq_small(bi_hbm, 2)
  ssq_big(pqi_hbm, 3)
  ssq_big(ywi_hbm, 4)
  pltpu.sync_copy(pbuf, reg_out.at[pl.ds(w * 8 * L, 8 * L)])


@functools.partial(
    pl.kernel,
    out_type=jax.ShapeDtypeStruct((NI, D), jnp.float32),
    mesh=_MESH,
    scratch_types=[
        pltpu.VMEM((RK,), jnp.float32),
        pltpu.VMEM((RK,), jnp.float32),
        pltpu.VMEM((RK, D), jnp.float32),
    ],
)
def _k2a_scale(ywi_hbm, hist_hbm, syw_out, h0v, h1v, rows_v):
  c = lax.axis_index("c")
  s = lax.axis_index("s")
  w = c * NS + s
  iot = lax.iota(jnp.int32, L)

  def chunk(i, _):
    cidx = w + NW * i

    @pl.when(cidx < NCH_ROWS)
    def _():
      r0 = cidx * RK
      pltpu.sync_copy(hist_hbm.at[pl.ds(r0, RK)], h0v)
      pltpu.sync_copy(hist_hbm.at[pl.ds(NI_PAD + r0, RK)], h1v)
      pltpu.sync_copy(ywi_hbm.at[pl.ds(r0, RK)], rows_v)

      def grp(g, _2):
        h = h0v[pl.ds(g * L, L)] + h1v[pl.ds(g * L, L)]
        n = 1.0 / jnp.maximum(h, 1.0)

        def rloop(r2, _3):
          nr = _dyn_gather(n, jnp.broadcast_to(r2, (L,)))
          r = g * L + r2
          for f in range(D // L):
            sl = pl.ds(f * L, L)
            rows_v[r, sl] = rows_v[r, sl] * nr
          return 0

        lax.fori_loop(0, L, rloop, 0)
        return 0

      lax.fori_loop(0, RK // L, grp, 0)
      pltpu.sync_copy(rows_v, syw_out.at[pl.ds(r0, RK)])

    return 0

  lax.fori_loop(0, (NCH_ROWS + NW - 1) // NW, chunk, 0)


@functools.partial(
    pl.kernel,
    out_type=jax.ShapeDtypeStruct((NC, NU, D), jnp.float32),
    mesh=_MESH,
    scratch_types=[
        pltpu.VMEM((3, EK), jnp.int32),
        pltpu.VMEM((2, EK), jnp.int32),
        pltpu.VMEM((2, EK, D), jnp.float32),
        pltpu.VMEM((16, D), jnp.float32),
        pltpu.VMEM_SHARED((NU, D), jnp.float32),
        pltpu.SemaphoreType.DMA((3,)),
        pltpu.SemaphoreType.DMA((2,)),
        pltpu.SemaphoreType.DMA((2,)),
        pltpu.SemaphoreType.DMA((2,)),
    ],
)
def _k2b_segsum(rate_hbm, syw_hbm, acc_out, ivs, ivd, rows4, zbuf, acc_sh,
                issem, idsem, gsem, ssem):
  c = lax.axis_index("c")
  s = lax.axis_index("s")
  w = c * NS + s
  zeros16 = jnp.zeros((L,), jnp.float32)

  for r in range(16):
    for f in range(D // L):
      zbuf[r, pl.ds(f * L, L)] = zeros16

  n_grp = NU // 16

  def zloop(i, _):
    g16 = s + NS * i

    @pl.when(g16 < n_grp)
    def _():
      pltpu.sync_copy(zbuf, acc_sh.at[pl.ds(g16 * 16, 16)])

    return 0

  lax.fori_loop(0, (n_grp + NS - 1) // NS, zloop, 0)
  plsc.subcore_barrier()

  base_s = w * E_TILE
  base_d = E_RATE + w * E_TILE

  def isrc_start(j, k):
    pltpu.async_copy(rate_hbm.at[pl.ds(base_s + j * EK, EK)], ivs.at[k],
                     issem.at[k])

  def isrc_wait(j, k):
    pltpu.make_async_copy(rate_hbm.at[pl.ds(base_s + j * EK, EK)], ivs.at[k],
                          issem.at[k]).wait()

  def idst_start(j, k):
    pltpu.async_copy(rate_hbm.at[pl.ds(base_d + j * EK, EK)], ivd.at[k],
                     idsem.at[k])

  def idst_wait(j, k):
    pltpu.make_async_copy(rate_hbm.at[pl.ds(base_d + j * EK, EK)], ivd.at[k],
                          idsem.at[k]).wait()

  def gather_start(j, b):
    pltpu.async_copy(syw_hbm.at[ivd.at[lax.rem(j, 2)]], rows4.at[b],
                     gsem.at[b])

  def gather_wait(j, b):
    pltpu.make_async_copy(syw_hbm.at[ivd.at[lax.rem(j, 2)]], rows4.at[b],
                          gsem.at[b]).wait()

  def scat_start(j, b):
    pltpu.async_copy(rows4.at[b], acc_sh.at[ivs.at[lax.rem(j, 3)]],
                     ssem.at[b], add=True)

  def scat_wait(j, b):
    pltpu.make_async_copy(rows4.at[b], acc_sh.at[ivs.at[lax.rem(j, 3)]],
                          ssem.at[b]).wait()

  idst_start(0, 0)
  idst_start(1, 1)
  isrc_start(0, 0)
  isrc_start(1, 1)
  isrc_start(2, 2)
  idst_wait(0, 0)
  gather_start(0, 0)

  def eloop(j, _):
    b = lax.rem(j, 2)
    gather_wait(j, b)
    isrc_wait(j, lax.rem(j, 3))
    scat_start(j, b)

    @pl.when(j >= 1)
    def _():
      scat_wait(j - 1, 1 - b)

    @pl.when(j + 2 < NCH_RATE)
    def _():
      isrc_start(j + 2, lax.rem(j + 2, 3))

    @pl.when(j + 1 < NCH_RATE)
    def _():
      idst_wait(j + 1, 1 - b)
      gather_start(j + 1, 1 - b)

    @pl.when(j + 2 < NCH_RATE)
    def _():
      idst_start(j + 2, b)

    return 0

  lax.fori_loop(0, NCH_RATE, eloop, 0)
  scat_wait(NCH_RATE - 1, lax.rem(NCH_RATE - 1, 2))
  plsc.subcore_barrier()

  def wloop(i, _):
    g16 = s + NS * i

    @pl.when(g16 < n_grp)
    def _():
      r0 = g16 * 16
      pltpu.sync_copy(acc_sh.at[pl.ds(r0, 16)], acc_out.at[c, pl.ds(r0, 16)])

    return 0

  lax.fori_loop(0, (n_grp + NS - 1) // NS, wloop, 0)


@functools.partial(
    pl.kernel,
    out_type=jax.ShapeDtypeStruct((NU, D), jnp.float32),
    mesh=_MESH,
    scratch_types=[
        pltpu.VMEM((RK, D), jnp.float32),
        pltpu.VMEM((RK, D), jnp.float32),
        pltpu.VMEM((RK, D), jnp.float32),
        pltpu.VMEM((RK, D), jnp.float32),
    ],
)
def _k2c_hu(acc_hbm, pqu_hbm, hu_out, a0v, a1v, pv, hv):
  c = lax.axis_index("c")
  s = lax.axis_index("s")
  w = c * NS + s

  def chunk(i, _):
    cidx = w + NW * i

    @pl.when(cidx < NCH_ROWS)
    def _():
      r0 = cidx * RK
      pltpu.sync_copy(acc_hbm.at[0, pl.ds(r0, RK)], a0v)
      pltpu.sync_copy(acc_hbm.at[1, pl.ds(r0, RK)], a1v)
      pltpu.sync_copy(pqu_hbm.at[pl.ds(r0, RK)], pv)

      def rloop(r, _2):
        for f in range(D // L):
          sl = pl.ds(f * L, L)
          hv[r, sl] = a0v[r, sl] + a1v[r, sl] + pv[r, sl]
        return 0

      lax.fori_loop(0, RK, rloop, 0)
      pltpu.sync_copy(hv, hu_out.at[pl.ds(r0, RK)])

    return 0

  lax.fori_loop(0, (NCH_ROWS + NW - 1) // NW, chunk, 0)


@functools.partial(
    pl.kernel,
    out_type=(
        jax.ShapeDtypeStruct((EP_PAD,), jnp.float32),
        jax.ShapeDtypeStruct((EP_PAD,), jnp.float32),
    ),
    mesh=_MESH,
    scratch_types=[
        pltpu.VMEM((NCH_PRED, EK3), jnp.int32),
        pltpu.VMEM((NCH_PRED, EK3), jnp.int32),
        pltpu.VMEM((NB3, QK, D), jnp.float32),
        pltpu.VMEM((NB3, QK, D), jnp.float32),
        pltpu.VMEM((NB3, QK), jnp.float32),
        pltpu.VMEM((NB3, QK), jnp.float32),
        pltpu.VMEM((NB3, QK), jnp.float32),
        pltpu.SemaphoreType.DMA((NB3,)),
        pltpu.SemaphoreType.DMA((NB3,)),
        pltpu.SemaphoreType.DMA((NB3,)),
        pltpu.SemaphoreType.DMA((NB3,)),
        pltpu.SemaphoreType.DMA((NB3,)),
    ],
)
def _k3_score(pos_hbm, neg_hbm, hu_hbm, pqi_hbm, bu_hbm, bi_hbm,
              pos_out, neg_out, ivs, ivd, uv2, pv2, bu2, bi2, sv2,
              usem, psem, busem, bisem, osem):
  c = lax.axis_index("c")
  s = lax.axis_index("s")
  w = c * NS + s
  iot = lax.iota(jnp.int32, L)
  qpr = EK3 // QK

  for edges_hbm, out_hbm in ((pos_hbm, pos_out), (neg_hbm, neg_out)):

    pltpu.sync_copy(edges_hbm.at[0, w], ivs)
    pltpu.sync_copy(edges_hbm.at[1, w], ivd)

    def sidx(j):
      return ivs.at[lax.div(j, qpr), pl.ds(lax.rem(j, qpr) * QK, QK)]

    def didx(j):
      return ivd.at[lax.div(j, qpr), pl.ds(lax.rem(j, qpr) * QK, QK)]

    def gathers_start(j, b):
      pltpu.async_copy(hu_hbm.at[sidx(j)], uv2.at[b], usem.at[b])
      pltpu.async_copy(pqi_hbm.at[didx(j)], pv2.at[b], psem.at[b])
      pltpu.async_copy(bu_hbm.at[sidx(j)], bu2.at[b], busem.at[b])
      pltpu.async_copy(bi_hbm.at[didx(j)], bi2.at[b], bisem.at[b])

    def gathers_wait(j, b):
      pltpu.make_async_copy(hu_hbm.at[sidx(j)], uv2.at[b],
                            usem.at[b]).wait()
      pltpu.make_async_copy(pqi_hbm.at[didx(j)], pv2.at[b],
                            psem.at[b]).wait()
      pltpu.make_async_copy(bu_hbm.at[sidx(j)], bu2.at[b],
                            busem.at[b]).wait()
      pltpu.make_async_copy(bi_hbm.at[didx(j)], bi2.at[b],
                            bisem.at[b]).wait()

    def out_start(j, b, out_hbm=out_hbm):
      pltpu.async_copy(sv2.at[b], out_hbm.at[pl.ds(w * EP_TILE + j * QK,
                                                   QK)], osem.at[b])

    def out_wait(j, b, out_hbm=out_hbm):
      pltpu.make_async_copy(sv2.at[b], out_hbm.at[pl.ds(w * EP_TILE + j * QK,
                                                        QK)],
                            osem.at[b]).wait()

    for b in range(NB3 - 1):
      gathers_start(b, b)

    def chunk(j, _):
      b = lax.rem(j, NB3)
      gathers_wait(j, b)

      @pl.when(j + NB3 - 1 < NQ)
      def _():
        gathers_start(j + NB3 - 1, lax.rem(j + NB3 - 1, NB3))

      @pl.when(j >= NB3)
      def _():
        out_wait(j - NB3, b)

      for g in range(QK // L):

        def edge_body(r2, merged, g=g, b=b):
          r = g * L + r2
          acc = jnp.zeros((L,), jnp.float32)
          for f in range(D // L):
            sl = pl.ds(f * L, L)
            acc = acc + uv2[b, r, sl] * pv2[b, r, sl]
          for sh in (8, 4, 2, 1):
            acc = acc + _dyn_gather(acc, jnp.bitwise_xor(iot, sh))
          return jnp.where(iot == r2, acc, merged)

        dots = lax.fori_loop(0, L, edge_body, jnp.zeros((L,), jnp.float32))
        sc = (dots + bu2[b, pl.ds(g * L, L)] + bi2[b, pl.ds(g * L, L)] + GB)
        sv2[b, pl.ds(g * L, L)] = sc
      out_start(j, b)
      return 0

    lax.fori_loop(0, NQ, chunk, 0)
    for dj in range(NB3):
      out_wait(NQ - NB3 + dj, lax.rem(NQ - NB3 + dj, NB3))


def kernel(rate_edges, trust_edges, pos_edges, neg_edges,
           pq_user, pq_item, yw_user, yw_item, b_user, b_item):
  del trust_edges, yw_user
  pqu_flat = pq_user.reshape(-1)
  pqi_flat = pq_item.reshape(-1)
  ywi_flat = yw_item.reshape(-1)
  bu_flat = b_user.reshape(-1)
  bi_flat = b_item.reshape(-1)

  rate4 = rate_edges.reshape(2, NW, NCH_RATE, EK)
  hist, regp = _k1_hist_reg(rate4, pqu_flat, pqi_flat, ywi_flat,
                            bu_flat, bi_flat)
  syw = _k2a_scale(yw_item, hist)
  acc = _k2b_segsum(rate_edges.reshape(-1), syw)
  h_u = _k2c_hu(acc, pq_user)

  pos4 = jnp.pad(pos_edges, ((0, 0), (0, EP_PAD - E_PRED))).reshape(
      2, NW, NCH_PRED, EK3)
  neg4 = jnp.pad(neg_edges, ((0, 0), (0, EP_PAD - E_PRED))).reshape(
      2, NW, NCH_PRED, EK3)
  pos_s, neg_s = _k3_score(pos4, neg4, h_u, pq_item, bu_flat, bi_flat)
  pos_score = pos_s[:E_PRED, None]
  neg_score = neg_s[:E_PRED, None]

  ssq = regp.reshape(NW, 8, L)[:, :5, :].sum(axis=(0, 2))
  nb_u, np_u, nb_i, np_i, ny_i = [jnp.sqrt(ssq[k]) for k in range(5)]
  reg_loss = ((LAM * E_RATE / NU) * (nb_u + np_u) +
              (LAM * E_RATE / NI) * (nb_i + np_i + ny_i))
  return (pos_score, neg_score, reg_loss)


# speedup vs baseline: 1.0292x; 1.0106x over previous
import functools

import jax
import jax.numpy as jnp
from jax import lax
from jax.experimental import pallas as pl
from jax.experimental.pallas import tpu as pltpu
from jax.experimental.pallas import tpu_sc as plsc

NU = 10000
NI = 10000
D = 128
LAM = 0.5
GB = 3.5
E_RATE = 320000
E_PRED = 100000

NC = 2
NS = 16
NW = NC * NS
L = 16

NI_PAD = 10240
E_TILE = E_RATE // NW
EK = 80
NCH_RATE = E_TILE // EK
RK = 80
NCH_ROWS = NU // RK
BIGC = 625
EP_PAD = 102400
EP_TILE = EP_PAD // NW
EK3 = 128
NCH_PRED = EP_TILE // EK3
QK = 64
NQ = EP_TILE // QK
NB3 = 6

_MESH = plsc.VectorSubcoreMesh(
    core_axis_name="c", subcore_axis_name="s", num_cores=NC, num_subcores=NS)


def _dyn_gather(x, idx):
  dnums = lax.GatherDimensionNumbers(
      offset_dims=(), collapsed_slice_dims=(0,), start_index_map=(0,))
  return lax.gather(x, idx[:, None], dnums, slice_sizes=(1,),
                    mode=lax.GatherScatterMode.PROMISE_IN_BOUNDS)


def _zero_fill(ref, n):
  zeros16 = jnp.zeros((L,), jnp.float32)

  def body(i, _):
    ref[pl.ds(i * L, L)] = zeros16
    return 0

  lax.fori_loop(0, n, body, 0)


@functools.partial(
    pl.kernel,
    out_type=(
        jax.ShapeDtypeStruct((NC * NI_PAD,), jnp.float32),
        jax.ShapeDtypeStruct((NW * 8 * L,), jnp.float32),
    ),
    mesh=_MESH,
    scratch_types=[
        pltpu.VMEM((NCH_RATE, EK), jnp.int32),
        pltpu.VMEM((EK,), jnp.float32),
        pltpu.VMEM((2048,), jnp.float32),
        pltpu.VMEM((8 * L,), jnp.float32),
        pltpu.VMEM_SHARED((NI_PAD,), jnp.float32),
        pltpu.SemaphoreType.DMA,
    ],
)
def _k1_hist_reg(rate_hbm, pqu_hbm, pqi_hbm, ywi_hbm, bu_hbm, bi_hbm,
                 hist_out, reg_out, ivd, ones_v, vbuf, pbuf, hist_sh, hsem):
  c = lax.axis_index("c")
  s = lax.axis_index("s")
  w = c * NS + s
  zeros16 = jnp.zeros((L,), jnp.float32)
  ones16 = jnp.ones((L,), jnp.float32)

  for g in range(EK // L):
    ones_v[pl.ds(g * L, L)] = ones16
  _zero_fill(vbuf, 2048 // L)
  pltpu.sync_copy(vbuf.at[pl.ds(0, NI_PAD // NS)],
                  hist_sh.at[pl.ds(s * (NI_PAD // NS), NI_PAD // NS)])
  pltpu.sync_copy(rate_hbm.at[1, w], ivd)
  plsc.subcore_barrier()

  def fire(j, _):
    pltpu.async_copy(ones_v, hist_sh.at[ivd.at[j]], hsem, add=True)
    return 0

  def drain(j, _):
    pltpu.make_async_copy(ones_v, hist_sh.at[ivd.at[j]], hsem).wait()
    return 0

  def grp(o, _):
    lax.fori_loop(o * 25, o * 25 + 25, fire, 0)
    lax.fori_loop(o * 25, o * 25 + 25, drain, 0)
    return 0

  lax.fori_loop(0, NCH_RATE // 25, grp, 0)
  plsc.subcore_barrier()
  pltpu.sync_copy(hist_sh.at[pl.ds(s * (NI_PAD // NS), NI_PAD // NS)],
                  hist_out.at[pl.ds(c * NI_PAD + s * (NI_PAD // NS),
                                    NI_PAD // NS)])

  _zero_fill(pbuf, 8)

  def ssq_big(arr_hbm, k):
    def chunk_body(i, _):
      cidx = w + NW * i

      @pl.when(cidx < BIGC)
      def _():
        pltpu.sync_copy(arr_hbm.at[pl.ds(cidx * 2048, 2048)], vbuf)

        def red(f, aa):
          v = vbuf[pl.ds(f * L, L)]
          return aa + v * v

        acc = lax.fori_loop(0, 2048 // L, red, zeros16)
        pbuf[pl.ds(k * L, L)] = pbuf[pl.ds(k * L, L)] + acc

      return 0

    lax.fori_loop(0, 20, chunk_body, 0)

  def ssq_small(arr_hbm, k):
    @pl.when(w < NU // 400)
    def _():
      pltpu.sync_copy(arr_hbm.at[pl.ds(w * 400, 400)], vbuf.at[pl.ds(0, 400)])

      def red(f, aa):
        v = vbuf[pl.ds(f * L, L)]
        return aa + v * v

      acc = lax.fori_loop(0, 400 // L, red, zeros16)
      pbuf[pl.ds(k * L, L)] = acc

  ssq_small(bu_hbm, 0)
  ssq_big(pqu_hbm, 1)
  ssq_small(bi_hbm, 2)
  ssq_big(pqi_hbm, 3)
  ssq_big(ywi_hbm, 4)
  pltpu.sync_copy(pbuf, reg_out.at[pl.ds(w * 8 * L, 8 * L)])


@functools.partial(
    pl.kernel,
    out_type=jax.ShapeDtypeStruct((NI, D), jnp.float32),
    mesh=_MESH,
    scratch_types=[
        pltpu.VMEM((RK,), jnp.float32),
        pltpu.VMEM((RK,), jnp.float32),
        pltpu.VMEM((RK, D), jnp.float32),
    ],
)
def _k2a_scale(ywi_hbm, hist_hbm, syw_out, h0v, h1v, rows_v):
  c = lax.axis_index("c")
  s = lax.axis_index("s")
  w = c * NS + s
  iot = lax.iota(jnp.int32, L)

  def chunk(i, _):
    cidx = w + NW * i

    @pl.when(cidx < NCH_ROWS)
    def _():
      r0 = cidx * RK
      pltpu.sync_copy(hist_hbm.at[pl.ds(r0, RK)], h0v)
      pltpu.sync_copy(hist_hbm.at[pl.ds(NI_PAD + r0, RK)], h1v)
      pltpu.sync_copy(ywi_hbm.at[pl.ds(r0, RK)], rows_v)

      def grp(g, _2):
        h = h0v[pl.ds(g * L, L)] + h1v[pl.ds(g * L, L)]
        n = 1.0 / jnp.maximum(h, 1.0)

        def rloop(r2, _3):
          nr = _dyn_gather(n, jnp.broadcast_to(r2, (L,)))
          r = g * L + r2
          for f in range(D // L):
            sl = pl.ds(f * L, L)
            rows_v[r, sl] = rows_v[r, sl] * nr
          return 0

        lax.fori_loop(0, L, rloop, 0)
        return 0

      lax.fori_loop(0, RK // L, grp, 0)
      pltpu.sync_copy(rows_v, syw_out.at[pl.ds(r0, RK)])

    return 0

  lax.fori_loop(0, (NCH_ROWS + NW - 1) // NW, chunk, 0)


@functools.partial(
    pl.kernel,
    out_type=jax.ShapeDtypeStruct((NC, NU, D), jnp.float32),
    mesh=_MESH,
    scratch_types=[
        pltpu.VMEM((3, EK), jnp.int32),
        pltpu.VMEM((2, EK), jnp.int32),
        pltpu.VMEM((2, EK, D), jnp.float32),
        pltpu.VMEM((16, D), jnp.float32),
        pltpu.VMEM_SHARED((NU, D), jnp.float32),
        pltpu.SemaphoreType.DMA((3,)),
        pltpu.SemaphoreType.DMA((2,)),
        pltpu.SemaphoreType.DMA((2,)),
        pltpu.SemaphoreType.DMA((2,)),
    ],
)
def _k2b_segsum(rate_hbm, syw_hbm, acc_out, ivs, ivd, rows4, zbuf, acc_sh,
                issem, idsem, gsem, ssem):
  c = lax.axis_index("c")
  s = lax.axis_index("s")
  w = c * NS + s
  zeros16 = jnp.zeros((L,), jnp.float32)

  for r in range(16):
    for f in range(D // L):
      zbuf[r, pl.ds(f * L, L)] = zeros16

  n_grp = NU // 16

  def zloop(i, _):
    g16 = s + NS * i

    @pl.when(g16 < n_grp)
    def _():
      pltpu.sync_copy(zbuf, acc_sh.at[pl.ds(g16 * 16, 16)])

    return 0

  lax.fori_loop(0, (n_grp + NS - 1) // NS, zloop, 0)
  plsc.subcore_barrier()

  base_s = w * E_TILE
  base_d = E_RATE + w * E_TILE

  def isrc_start(j, k):
    pltpu.async_copy(rate_hbm.at[pl.ds(base_s + j * EK, EK)], ivs.at[k],
                     issem.at[k])

  def isrc_wait(j, k):
    pltpu.make_async_copy(rate_hbm.at[pl.ds(base_s + j * EK, EK)], ivs.at[k],
                          issem.at[k]).wait()

  def idst_start(j, k):
    pltpu.async_copy(rate_hbm.at[pl.ds(base_d + j * EK, EK)], ivd.at[k],
                     idsem.at[k])

  def idst_wait(j, k):
    pltpu.make_async_copy(rate_hbm.at[pl.ds(base_d + j * EK, EK)], ivd.at[k],
                          idsem.at[k]).wait()

  def gather_start(j, b):
    pltpu.async_copy(syw_hbm.at[ivd.at[lax.rem(j, 2)]], rows4.at[b],
                     gsem.at[b])

  def gather_wait(j, b):
    pltpu.make_async_copy(syw_hbm.at[ivd.at[lax.rem(j, 2)]], rows4.at[b],
                          gsem.at[b]).wait()

  def scat_start(j, b):
    pltpu.async_copy(rows4.at[b], acc_sh.at[ivs.at[lax.rem(j, 3)]],
                     ssem.at[b], add=True)

  def scat_wait(j, b):
    pltpu.make_async_copy(rows4.at[b], acc_sh.at[ivs.at[lax.rem(j, 3)]],
                          ssem.at[b]).wait()

  idst_start(0, 0)
  idst_start(1, 1)
  isrc_start(0, 0)
  isrc_start(1, 1)
  isrc_start(2, 2)
  idst_wait(0, 0)
  gather_start(0, 0)

  def eloop(j, _):
    b = lax.rem(j, 2)
    gather_wait(j, b)
    isrc_wait(j, lax.rem(j, 3))
    scat_start(j, b)

    @pl.when(j >= 1)
    def _():
      scat_wait(j - 1, 1 - b)

    @pl.when(j + 2 < NCH_RATE)
    def _():
      isrc_start(j + 2, lax.rem(j + 2, 3))

    @pl.when(j + 1 < NCH_RATE)
    def _():
      idst_wait(j + 1, 1 - b)
      gather_start(j + 1, 1 - b)

    @pl.when(j + 2 < NCH_RATE)
    def _():
      idst_start(j + 2, b)

    return 0

  lax.fori_loop(0, NCH_RATE, eloop, 0)
  scat_wait(NCH_RATE - 1, lax.rem(NCH_RATE - 1, 2))
  plsc.subcore_barrier()

  def wloop(i, _):
    g16 = s + NS * i

    @pl.when(g16 < n_grp)
    def _():
      r0 = g16 * 16
      pltpu.sync_copy(acc_sh.at[pl.ds(r0, 16)], acc_out.at[c, pl.ds(r0, 16)])

    return 0

  lax.fori_loop(0, (n_grp + NS - 1) // NS, wloop, 0)


@functools.partial(
    pl.kernel,
    out_type=jax.ShapeDtypeStruct((NU, D), jnp.float32),
    mesh=_MESH,
    scratch_types=[
        pltpu.VMEM((RK, D), jnp.float32),
        pltpu.VMEM((RK, D), jnp.float32),
        pltpu.VMEM((RK, D), jnp.float32),
        pltpu.VMEM((RK, D), jnp.float32),
    ],
)
def _k2c_hu(acc_hbm, pqu_hbm, hu_out, a0v, a1v, pv, hv):
  c = lax.axis_index("c")
  s = lax.axis_index("s")
  w = c * NS + s

  def chunk(i, _):
    cidx = w + NW * i

    @pl.when(cidx < NCH_ROWS)
    def _():
      r0 = cidx * RK
      pltpu.sync_copy(acc_hbm.at[0, pl.ds(r0, RK)], a0v)
      pltpu.sync_copy(acc_hbm.at[1, pl.ds(r0, RK)], a1v)
      pltpu.sync_copy(pqu_hbm.at[pl.ds(r0, RK)], pv)

      def rloop(r, _2):
        for f in range(D // L):
          sl = pl.ds(f * L, L)
          hv[r, sl] = a0v[r, sl] + a1v[r, sl] + pv[r, sl]
        return 0

      lax.fori_loop(0, RK, rloop, 0)
      pltpu.sync_copy(hv, hu_out.at[pl.ds(r0, RK)])

    return 0

  lax.fori_loop(0, (NCH_ROWS + NW - 1) // NW, chunk, 0)


@functools.partial(
    pl.kernel,
    out_type=(
        jax.ShapeDtypeStruct((EP_PAD,), jnp.float32),
        jax.ShapeDtypeStruct((EP_PAD,), jnp.float32),
    ),
    mesh=_MESH,
    scratch_types=[
        pltpu.VMEM((NCH_PRED, EK3), jnp.int32),
        pltpu.VMEM((NCH_PRED, EK3), jnp.int32),
        pltpu.VMEM((NB3, QK, D), jnp.float32),
        pltpu.VMEM((NB3, QK, D), jnp.float32),
        pltpu.VMEM((NB3, QK), jnp.float32),
        pltpu.VMEM((NB3, QK), jnp.float32),
        pltpu.VMEM((EP_TILE,), jnp.float32),
        pltpu.SemaphoreType.DMA((NB3,)),
        pltpu.SemaphoreType.DMA((NB3,)),
        pltpu.SemaphoreType.DMA((NB3,)),
        pltpu.SemaphoreType.DMA((NB3,)),
    ],
)
def _k3_score(pos_hbm, neg_hbm, hu_hbm, pqi_hbm, bu_hbm, bi_hbm,
              pos_out, neg_out, ivs, ivd, uv2, pv2, bu2, bi2, sv_all,
              usem, psem, busem, bisem):
  c = lax.axis_index("c")
  s = lax.axis_index("s")
  w = c * NS + s
  iot = lax.iota(jnp.int32, L)
  qpr = EK3 // QK

  def run_set(edges_hbm, out_hbm, w):
    pltpu.sync_copy(edges_hbm.at[0, w], ivs)
    pltpu.sync_copy(edges_hbm.at[1, w], ivd)

    def sidx(j):
      return ivs.at[lax.div(j, qpr), pl.ds(lax.rem(j, qpr) * QK, QK)]

    def didx(j):
      return ivd.at[lax.div(j, qpr), pl.ds(lax.rem(j, qpr) * QK, QK)]

    def gathers_start(j, b):
      pltpu.async_copy(hu_hbm.at[sidx(j)], uv2.at[b], usem.at[b])
      pltpu.async_copy(pqi_hbm.at[didx(j)], pv2.at[b], psem.at[b])
      pltpu.async_copy(bu_hbm.at[sidx(j)], bu2.at[b], busem.at[b])
      pltpu.async_copy(bi_hbm.at[didx(j)], bi2.at[b], bisem.at[b])

    def gathers_wait(j, b):
      pltpu.make_async_copy(hu_hbm.at[sidx(j)], uv2.at[b],
                            usem.at[b]).wait()
      pltpu.make_async_copy(pqi_hbm.at[didx(j)], pv2.at[b],
                            psem.at[b]).wait()
      pltpu.make_async_copy(bu_hbm.at[sidx(j)], bu2.at[b],
                            busem.at[b]).wait()
      pltpu.make_async_copy(bi_hbm.at[didx(j)], bi2.at[b],
                            bisem.at[b]).wait()

    for b in range(NB3 - 1):
      gathers_start(b, b)

    def chunk(j, _):
      b = lax.rem(j, NB3)
      gathers_wait(j, b)

      @pl.when(j + NB3 - 1 < NQ)
      def _():
        gathers_start(j + NB3 - 1, lax.rem(j + NB3 - 1, NB3))

      for g in range(QK // L):

        def edge_body(r2, merged, g=g, b=b):
          r = g * L + r2
          acc = jnp.zeros((L,), jnp.float32)
          for f in range(D // L):
            sl = pl.ds(f * L, L)
            acc = acc + uv2[b, r, sl] * pv2[b, r, sl]
          for sh in (8, 4, 2, 1):
            acc = acc + _dyn_gather(acc, jnp.bitwise_xor(iot, sh))
          return jnp.where(iot == r2, acc, merged)

        dots = lax.fori_loop(0, L, edge_body, jnp.zeros((L,), jnp.float32))
        sc = (dots + bu2[b, pl.ds(g * L, L)] + bi2[b, pl.ds(g * L, L)] + GB)
        sv_all[pl.ds(j * QK + g * L, L)] = sc
      return 0

    lax.fori_loop(0, NQ, chunk, 0)
    pltpu.sync_copy(sv_all, out_hbm.at[pl.ds(w * EP_TILE, EP_TILE)])

  for edges_hbm, out_hbm in ((pos_hbm, pos_out), (neg_hbm, neg_out)):
    run_set(edges_hbm, out_hbm, w)


def kernel(rate_edges, trust_edges, pos_edges, neg_edges,
           pq_user, pq_item, yw_user, yw_item, b_user, b_item):
  del trust_edges, yw_user
  pqu_flat = pq_user.reshape(-1)
  pqi_flat = pq_item.reshape(-1)
  ywi_flat = yw_item.reshape(-1)
  bu_flat = b_user.reshape(-1)
  bi_flat = b_item.reshape(-1)

  rate4 = rate_edges.reshape(2, NW, NCH_RATE, EK)
  hist, regp = _k1_hist_reg(rate4, pqu_flat, pqi_flat, ywi_flat,
                            bu_flat, bi_flat)
  syw = _k2a_scale(yw_item, hist)
  acc = _k2b_segsum(rate_edges.reshape(-1), syw)
  h_u = _k2c_hu(acc, pq_user)

  pos4 = jnp.pad(pos_edges, ((0, 0), (0, EP_PAD - E_PRED))).reshape(
      2, NW, NCH_PRED, EK3)
  neg4 = jnp.pad(neg_edges, ((0, 0), (0, EP_PAD - E_PRED))).reshape(
      2, NW, NCH_PRED, EK3)
  pos_s, neg_s = _k3_score(pos4, neg4, h_u, pq_item, bu_flat, bi_flat)
  pos_score = pos_s[:E_PRED, None]
  neg_score = neg_s[:E_PRED, None]

  ssq = regp.reshape(NW, 8, L)[:, :5, :].sum(axis=(0, 2))
  nb_u, np_u, nb_i, np_i, ny_i = [jnp.sqrt(ssq[k]) for k in range(5)]
  reg_loss = ((LAM * E_RATE / NU) * (nb_u + np_u) +
              (LAM * E_RATE / NI) * (nb_i + np_i + ny_i))
  return (pos_score, neg_score, reg_loss)


# speedup vs baseline: 1.0905x; 1.0596x over previous
import functools

import jax
import jax.numpy as jnp
from jax import lax
from jax.experimental import pallas as pl
from jax.experimental.pallas import tpu as pltpu
from jax.experimental.pallas import tpu_sc as plsc

NU = 10000
NI = 10000
D = 128
LAM = 0.5
GB = 3.5
E_RATE = 320000
E_PRED = 100000

NC = 2
NS = 16
NW = NC * NS
L = 16

NI_PAD = 10240
E_TILE = E_RATE // NW
EK = 80
NCH_RATE = E_TILE // EK
RK = 80
NCH_ROWS = NU // RK
BIGC = 625
EP_PAD = 102400
EP_TILE = EP_PAD // NW
EK3 = 128
NCH_PRED = EP_TILE // EK3
QK = 64
NB3 = 6
E0 = 4224
E1 = EP_TILE * 2 - E0

_MESH = plsc.VectorSubcoreMesh(
    core_axis_name="c", subcore_axis_name="s", num_cores=NC, num_subcores=NS)


def _dyn_gather(x, idx):
  dnums = lax.GatherDimensionNumbers(
      offset_dims=(), collapsed_slice_dims=(0,), start_index_map=(0,))
  return lax.gather(x, idx[:, None], dnums, slice_sizes=(1,),
                    mode=lax.GatherScatterMode.PROMISE_IN_BOUNDS)


def _zero_fill(ref, n):
  zeros16 = jnp.zeros((L,), jnp.float32)

  def body(i, _):
    ref[pl.ds(i * L, L)] = zeros16
    return 0

  lax.fori_loop(0, n, body, 0)


@functools.partial(
    pl.kernel,
    out_type=(
        jax.ShapeDtypeStruct((NC * NI_PAD,), jnp.float32),
        jax.ShapeDtypeStruct((NW * 8 * L,), jnp.float32),
    ),
    mesh=_MESH,
    scratch_types=[
        pltpu.VMEM((NCH_RATE, EK), jnp.int32),
        pltpu.VMEM((EK,), jnp.float32),
        pltpu.VMEM((2048,), jnp.float32),
        pltpu.VMEM((8 * L,), jnp.float32),
        pltpu.VMEM_SHARED((NI_PAD,), jnp.float32),
        pltpu.SemaphoreType.DMA,
    ],
)
def _k1_hist_reg(rate_hbm, pqu_hbm, pqi_hbm, ywi_hbm, bu_hbm, bi_hbm,
                 hist_out, reg_out, ivd, ones_v, vbuf, pbuf, hist_sh, hsem):
  c = lax.axis_index("c")
  s = lax.axis_index("s")
  w = c * NS + s
  zeros16 = jnp.zeros((L,), jnp.float32)
  ones16 = jnp.ones((L,), jnp.float32)

  for g in range(EK // L):
    ones_v[pl.ds(g * L, L)] = ones16
  _zero_fill(vbuf, 2048 // L)
  pltpu.sync_copy(vbuf.at[pl.ds(0, NI_PAD // NS)],
                  hist_sh.at[pl.ds(s * (NI_PAD // NS), NI_PAD // NS)])
  pltpu.sync_copy(rate_hbm.at[1, w], ivd)
  plsc.subcore_barrier()

  def fire(j, _):
    pltpu.async_copy(ones_v, hist_sh.at[ivd.at[j]], hsem, add=True)
    return 0

  def drain(j, _):
    pltpu.make_async_copy(ones_v, hist_sh.at[ivd.at[j]], hsem).wait()
    return 0

  def grp(o, _):
    lax.fori_loop(o * 25, o * 25 + 25, fire, 0)
    lax.fori_loop(o * 25, o * 25 + 25, drain, 0)
    return 0

  lax.fori_loop(0, NCH_RATE // 25, grp, 0)
  plsc.subcore_barrier()
  pltpu.sync_copy(hist_sh.at[pl.ds(s * (NI_PAD // NS), NI_PAD // NS)],
                  hist_out.at[pl.ds(c * NI_PAD + s * (NI_PAD // NS),
                                    NI_PAD // NS)])

  _zero_fill(pbuf, 8)

  def ssq_big(arr_hbm, k):
    def chunk_body(i, _):
      cidx = w + NW * i

      @pl.when(cidx < BIGC)
      def _():
        pltpu.sync_copy(arr_hbm.at[pl.ds(cidx * 2048, 2048)], vbuf)

        def red(f, aa):
          v = vbuf[pl.ds(f * L, L)]
          return aa + v * v

        acc = lax.fori_loop(0, 2048 // L, red, zeros16)
        pbuf[pl.ds(k * L, L)] = pbuf[pl.ds(k * L, L)] + acc

      return 0

    lax.fori_loop(0, 20, chunk_body, 0)

  def ssq_small(arr_hbm, k):
    @pl.when(w < NU // 400)
    def _():
      pltpu.sync_copy(arr_hbm.at[pl.ds(w * 400, 400)], vbuf.at[pl.ds(0, 400)])

      def red(f, aa):
        v = vbuf[pl.ds(f * L, L)]
        return aa + v * v

      acc = lax.fori_loop(0, 400 // L, red, zeros16)
      pbuf[pl.ds(k * L, L)] = acc

  ssq_small(bu_hbm, 0)
  ssq_big(pqu_hbm, 1)
  ssq_small(bi_hbm, 2)
  ssq_big(pqi_hbm, 3)
  ssq_big(ywi_hbm, 4)
  pltpu.sync_copy(pbuf, reg_out.at[pl.ds(w * 8 * L, 8 * L)])


@functools.partial(
    pl.kernel,
    out_type=jax.ShapeDtypeStruct((NI, D), jnp.float32),
    mesh=_MESH,
    scratch_types=[
        pltpu.VMEM((RK,), jnp.float32),
        pltpu.VMEM((RK,), jnp.float32),
        pltpu.VMEM((RK, D), jnp.float32),
    ],
)
def _k2a_scale(ywi_hbm, hist_hbm, syw_out, h0v, h1v, rows_v):
  c = lax.axis_index("c")
  s = lax.axis_index("s")
  w = c * NS + s
  iot = lax.iota(jnp.int32, L)

  def chunk(i, _):
    cidx = w + NW * i

    @pl.when(cidx < NCH_ROWS)
    def _():
      r0 = cidx * RK
      pltpu.sync_copy(hist_hbm.at[pl.ds(r0, RK)], h0v)
      pltpu.sync_copy(hist_hbm.at[pl.ds(NI_PAD + r0, RK)], h1v)
      pltpu.sync_copy(ywi_hbm.at[pl.ds(r0, RK)], rows_v)

      def grp(g, _2):
        h = h0v[pl.ds(g * L, L)] + h1v[pl.ds(g * L, L)]
        n = 1.0 / jnp.maximum(h, 1.0)

        def rloop(r2, _3):
          nr = _dyn_gather(n, jnp.broadcast_to(r2, (L,)))
          r = g * L + r2
          for f in range(D // L):
            sl = pl.ds(f * L, L)
            rows_v[r, sl] = rows_v[r, sl] * nr
          return 0

        lax.fori_loop(0, L, rloop, 0)
        return 0

      lax.fori_loop(0, RK // L, grp, 0)
      pltpu.sync_copy(rows_v, syw_out.at[pl.ds(r0, RK)])

    return 0

  lax.fori_loop(0, (NCH_ROWS + NW - 1) // NW, chunk, 0)


@functools.partial(
    pl.kernel,
    out_type=jax.ShapeDtypeStruct((NC, NU, D), jnp.float32),
    mesh=_MESH,
    scratch_types=[
        pltpu.VMEM((3, EK), jnp.int32),
        pltpu.VMEM((2, EK), jnp.int32),
        pltpu.VMEM((2, EK, D), jnp.float32),
        pltpu.VMEM((16, D), jnp.float32),
        pltpu.VMEM_SHARED((NU, D), jnp.float32),
        pltpu.SemaphoreType.DMA((3,)),
        pltpu.SemaphoreType.DMA((2,)),
        pltpu.SemaphoreType.DMA((2,)),
        pltpu.SemaphoreType.DMA((2,)),
    ],
)
def _k2b_segsum(rate_hbm, syw_hbm, acc_out, ivs, ivd, rows4, zbuf, acc_sh,
                issem, idsem, gsem, ssem):
  c = lax.axis_index("c")
  s = lax.axis_index("s")
  w = c * NS + s
  zeros16 = jnp.zeros((L,), jnp.float32)

  for r in range(16):
    for f in range(D // L):
      zbuf[r, pl.ds(f * L, L)] = zeros16

  n_grp = NU // 16

  def zloop(i, _):
    g16 = s + NS * i

    @pl.when(g16 < n_grp)
    def _():
      pltpu.sync_copy(zbuf, acc_sh.at[pl.ds(g16 * 16, 16)])

    return 0

  lax.fori_loop(0, (n_grp + NS - 1) // NS, zloop, 0)
  plsc.subcore_barrier()

  base_s = w * E_TILE
  base_d = E_RATE + w * E_TILE

  def isrc_start(j, k):
    pltpu.async_copy(rate_hbm.at[pl.ds(base_s + j * EK, EK)], ivs.at[k],
                     issem.at[k])

  def isrc_wait(j, k):
    pltpu.make_async_copy(rate_hbm.at[pl.ds(base_s + j * EK, EK)], ivs.at[k],
                          issem.at[k]).wait()

  def idst_start(j, k):
    pltpu.async_copy(rate_hbm.at[pl.ds(base_d + j * EK, EK)], ivd.at[k],
                     idsem.at[k])

  def idst_wait(j, k):
    pltpu.make_async_copy(rate_hbm.at[pl.ds(base_d + j * EK, EK)], ivd.at[k],
                          idsem.at[k]).wait()

  def gather_start(j, b):
    pltpu.async_copy(syw_hbm.at[ivd.at[lax.rem(j, 2)]], rows4.at[b],
                     gsem.at[b])

  def gather_wait(j, b):
    pltpu.make_async_copy(syw_hbm.at[ivd.at[lax.rem(j, 2)]], rows4.at[b],
                          gsem.at[b]).wait()

  def scat_start(j, b):
    pltpu.async_copy(rows4.at[b], acc_sh.at[ivs.at[lax.rem(j, 3)]],
                     ssem.at[b], add=True)

  def scat_wait(j, b):
    pltpu.make_async_copy(rows4.at[b], acc_sh.at[ivs.at[lax.rem(j, 3)]],
                          ssem.at[b]).wait()

  idst_start(0, 0)
  idst_start(1, 1)
  isrc_start(0, 0)
  isrc_start(1, 1)
  isrc_start(2, 2)
  idst_wait(0, 0)
  gather_start(0, 0)

  def eloop(j, _):
    b = lax.rem(j, 2)
    gather_wait(j, b)
    isrc_wait(j, lax.rem(j, 3))
    scat_start(j, b)

    @pl.when(j >= 1)
    def _():
      scat_wait(j - 1, 1 - b)

    @pl.when(j + 2 < NCH_RATE)
    def _():
      isrc_start(j + 2, lax.rem(j + 2, 3))

    @pl.when(j + 1 < NCH_RATE)
    def _():
      idst_wait(j + 1, 1 - b)
      gather_start(j + 1, 1 - b)

    @pl.when(j + 2 < NCH_RATE)
    def _():
      idst_start(j + 2, b)

    return 0

  lax.fori_loop(0, NCH_RATE, eloop, 0)
  scat_wait(NCH_RATE - 1, lax.rem(NCH_RATE - 1, 2))
  plsc.subcore_barrier()

  def wloop(i, _):
    g16 = s + NS * i

    @pl.when(g16 < n_grp)
    def _():
      r0 = g16 * 16
      pltpu.sync_copy(acc_sh.at[pl.ds(r0, 16)], acc_out.at[c, pl.ds(r0, 16)])

    return 0

  lax.fori_loop(0, (n_grp + NS - 1) // NS, wloop, 0)


@functools.partial(
    pl.kernel,
    out_type=jax.ShapeDtypeStruct((NU, D), jnp.float32),
    mesh=_MESH,
    scratch_types=[
        pltpu.VMEM((RK, D), jnp.float32),
        pltpu.VMEM((RK, D), jnp.float32),
        pltpu.VMEM((RK, D), jnp.float32),
        pltpu.VMEM((RK, D), jnp.float32),
    ],
)
def _k2c_hu(acc_hbm, pqu_hbm, hu_out, a0v, a1v, pv, hv):
  c = lax.axis_index("c")
  s = lax.axis_index("s")
  w = c * NS + s

  def chunk(i, _):
    cidx = w + NW * i

    @pl.when(cidx < NCH_ROWS)
    def _():
      r0 = cidx * RK
      pltpu.sync_copy(acc_hbm.at[0, pl.ds(r0, RK)], a0v)
      pltpu.sync_copy(acc_hbm.at[1, pl.ds(r0, RK)], a1v)
      pltpu.sync_copy(pqu_hbm.at[pl.ds(r0, RK)], pv)

      def rloop(r, _2):
        for f in range(D // L):
          sl = pl.ds(f * L, L)
          hv[r, sl] = a0v[r, sl] + a1v[r, sl] + pv[r, sl]
        return 0

      lax.fori_loop(0, RK, rloop, 0)
      pltpu.sync_copy(hv, hu_out.at[pl.ds(r0, RK)])

    return 0

  lax.fori_loop(0, (NCH_ROWS + NW - 1) // NW, chunk, 0)


@functools.partial(
    pl.kernel,
    out_type=(
        jax.ShapeDtypeStruct((EP_PAD,), jnp.float32),
        jax.ShapeDtypeStruct((EP_PAD,), jnp.float32),
    ),
    mesh=_MESH,
    scratch_types=[
        pltpu.VMEM((E0,), jnp.int32),
        pltpu.VMEM((E0,), jnp.int32),
        pltpu.VMEM((NB3, QK, D), jnp.float32),
        pltpu.VMEM((NB3, QK, D), jnp.float32),
        pltpu.VMEM((NB3, QK), jnp.float32),
        pltpu.VMEM((NB3, QK), jnp.float32),
        pltpu.VMEM((E0,), jnp.float32),
        pltpu.SemaphoreType.DMA((NB3,)),
        pltpu.SemaphoreType.DMA((NB3,)),
        pltpu.SemaphoreType.DMA((NB3,)),
        pltpu.SemaphoreType.DMA((NB3,)),
    ],
)
def _k3_score(pos_hbm, neg_hbm, hu_hbm, pqi_hbm, bu_hbm, bi_hbm,
              pos_out, neg_out, ivs, ivd, uv2, pv2, bu2, bi2, sv_all,
              usem, psem, busem, bisem):
  c = lax.axis_index("c")
  s = lax.axis_index("s")
  iot = lax.iota(jnp.int32, L)

  def run_set(edges_hbm, out_hbm, base, ne):
    nq = ne // QK
    pltpu.sync_copy(edges_hbm.at[pl.ds(base, ne)], ivs.at[pl.ds(0, ne)])
    pltpu.sync_copy(edges_hbm.at[pl.ds(EP_PAD + base, ne)],
                    ivd.at[pl.ds(0, ne)])

    def sidx(j):
      return ivs.at[pl.ds(j * QK, QK)]

    def didx(j):
      return ivd.at[pl.ds(j * QK, QK)]

    def gathers_start(j, b):
      pltpu.async_copy(hu_hbm.at[sidx(j)], uv2.at[b], usem.at[b])
      pltpu.async_copy(pqi_hbm.at[didx(j)], pv2.at[b], psem.at[b])
      pltpu.async_copy(bu_hbm.at[sidx(j)], bu2.at[b], busem.at[b])
      pltpu.async_copy(bi_hbm.at[didx(j)], bi2.at[b], bisem.at[b])

    def gathers_wait(j, b):
      pltpu.make_async_copy(hu_hbm.at[sidx(j)], uv2.at[b],
                            usem.at[b]).wait()
      pltpu.make_async_copy(pqi_hbm.at[didx(j)], pv2.at[b],
                            psem.at[b]).wait()
      pltpu.make_async_copy(bu_hbm.at[sidx(j)], bu2.at[b],
                            busem.at[b]).wait()
      pltpu.make_async_copy(bi_hbm.at[didx(j)], bi2.at[b],
                            bisem.at[b]).wait()

    for b in range(NB3 - 1):
      gathers_start(b, b)

    def chunk(j, _):
      b = lax.rem(j, NB3)
      gathers_wait(j, b)

      @pl.when(j + NB3 - 1 < nq)
      def _():
        gathers_start(j + NB3 - 1, lax.rem(j + NB3 - 1, NB3))

      for g in range(QK // L):

        def edge_body(r2, merged, g=g, b=b):
          r = g * L + r2
          acc = jnp.zeros((L,), jnp.float32)
          for f in range(D // L):
            sl = pl.ds(f * L, L)
            acc = acc + uv2[b, r, sl] * pv2[b, r, sl]
          for sh in (8, 4, 2, 1):
            acc = acc + _dyn_gather(acc, jnp.bitwise_xor(iot, sh))
          return jnp.where(iot == r2, acc, merged)

        dots = lax.fori_loop(0, L, edge_body, jnp.zeros((L,), jnp.float32))
        sc = (dots + bu2[b, pl.ds(g * L, L)] + bi2[b, pl.ds(g * L, L)] + GB)
        sv_all[pl.ds(j * QK + g * L, L)] = sc
      return 0

    lax.fori_loop(0, nq, chunk, 0)
    pltpu.sync_copy(sv_all.at[pl.ds(0, ne)], out_hbm.at[pl.ds(base, ne)])

  for edges_hbm, out_hbm in ((pos_hbm, pos_out), (neg_hbm, neg_out)):

    @pl.when(c == 0)
    def _(edges_hbm=edges_hbm, out_hbm=out_hbm):
      run_set(edges_hbm, out_hbm, s * (2 * EP_TILE), E0)

    @pl.when(c == 1)
    def _(edges_hbm=edges_hbm, out_hbm=out_hbm):
      run_set(edges_hbm, out_hbm, s * (2 * EP_TILE) + E0, E1)


def kernel(rate_edges, trust_edges, pos_edges, neg_edges,
           pq_user, pq_item, yw_user, yw_item, b_user, b_item):
  del trust_edges, yw_user
  pqu_flat = pq_user.reshape(-1)
  pqi_flat = pq_item.reshape(-1)
  ywi_flat = yw_item.reshape(-1)
  bu_flat = b_user.reshape(-1)
  bi_flat = b_item.reshape(-1)

  rate4 = rate_edges.reshape(2, NW, NCH_RATE, EK)
  hist, regp = _k1_hist_reg(rate4, pqu_flat, pqi_flat, ywi_flat,
                            bu_flat, bi_flat)
  syw = _k2a_scale(yw_item, hist)
  acc = _k2b_segsum(rate_edges.reshape(-1), syw)
  h_u = _k2c_hu(acc, pq_user)

  pos_f = jnp.pad(pos_edges, ((0, 0), (0, EP_PAD - E_PRED))).reshape(-1)
  neg_f = jnp.pad(neg_edges, ((0, 0), (0, EP_PAD - E_PRED))).reshape(-1)
  pos_s, neg_s = _k3_score(pos_f, neg_f, h_u, pq_item, bu_flat, bi_flat)
  pos_score = pos_s[:E_PRED, None]
  neg_score = neg_s[:E_PRED, None]

  ssq = regp.reshape(NW, 8, L)[:, :5, :].sum(axis=(0, 2))
  nb_u, np_u, nb_i, np_i, ny_i = [jnp.sqrt(ssq[k]) for k in range(5)]
  reg_loss = ((LAM * E_RATE / NU) * (nb_u + np_u) +
              (LAM * E_RATE / NI) * (nb_i + np_i + ny_i))
  return (pos_score, neg_score, reg_loss)


# speedup vs baseline: 1.1668x; 1.0700x over previous
import functools

import jax
import jax.numpy as jnp
from jax import lax
from jax.experimental import pallas as pl
from jax.experimental.pallas import tpu as pltpu
from jax.experimental.pallas import tpu_sc as plsc

NU = 10000
NI = 10000
D = 128
LAM = 0.5
GB = 3.5
E_RATE = 320000
E_PRED = 100000

NC = 2
NS = 16
NW = NC * NS
L = 16

NI_PAD = 10240
E_TILE = E_RATE // NW
EK = 80
NCH_RATE = E_TILE // EK
RK = 80
NCH_ROWS = NU // RK
BIGC = 625
EP_PAD = 102400
EP_TILE = EP_PAD // NW
EK3 = 128
NCH_PRED = EP_TILE // EK3
QK = 64
NB3 = 6
E0 = 4864
E1 = EP_TILE * 2 - E0

_MESH = plsc.VectorSubcoreMesh(
    core_axis_name="c", subcore_axis_name="s", num_cores=NC, num_subcores=NS)


def _dyn_gather(x, idx):
  dnums = lax.GatherDimensionNumbers(
      offset_dims=(), collapsed_slice_dims=(0,), start_index_map=(0,))
  return lax.gather(x, idx[:, None], dnums, slice_sizes=(1,),
                    mode=lax.GatherScatterMode.PROMISE_IN_BOUNDS)


def _zero_fill(ref, n):
  zeros16 = jnp.zeros((L,), jnp.float32)

  def body(i, _):
    ref[pl.ds(i * L, L)] = zeros16
    return 0

  lax.fori_loop(0, n, body, 0)


@functools.partial(
    pl.kernel,
    out_type=(
        jax.ShapeDtypeStruct((NC * NI_PAD,), jnp.float32),
        jax.ShapeDtypeStruct((NW * 8 * L,), jnp.float32),
    ),
    mesh=_MESH,
    scratch_types=[
        pltpu.VMEM((NCH_RATE, EK), jnp.int32),
        pltpu.VMEM((EK,), jnp.float32),
        pltpu.VMEM((2048,), jnp.float32),
        pltpu.VMEM((8 * L,), jnp.float32),
        pltpu.VMEM_SHARED((NI_PAD,), jnp.float32),
        pltpu.SemaphoreType.DMA,
    ],
)
def _k1_hist_reg(rate_hbm, pqu_hbm, pqi_hbm, ywi_hbm, bu_hbm, bi_hbm,
                 hist_out, reg_out, ivd, ones_v, vbuf, pbuf, hist_sh, hsem):
  c = lax.axis_index("c")
  s = lax.axis_index("s")
  w = c * NS + s
  zeros16 = jnp.zeros((L,), jnp.float32)
  ones16 = jnp.ones((L,), jnp.float32)

  for g in range(EK // L):
    ones_v[pl.ds(g * L, L)] = ones16
  _zero_fill(vbuf, 2048 // L)
  pltpu.sync_copy(vbuf.at[pl.ds(0, NI_PAD // NS)],
                  hist_sh.at[pl.ds(s * (NI_PAD // NS), NI_PAD // NS)])
  pltpu.sync_copy(rate_hbm.at[1, w], ivd)
  plsc.subcore_barrier()

  def fire(j, _):
    pltpu.async_copy(ones_v, hist_sh.at[ivd.at[j]], hsem, add=True)
    return 0

  def drain(j, _):
    pltpu.make_async_copy(ones_v, hist_sh.at[ivd.at[j]], hsem).wait()
    return 0

  def grp(o, _):
    lax.fori_loop(o * 25, o * 25 + 25, fire, 0)
    lax.fori_loop(o * 25, o * 25 + 25, drain, 0)
    return 0

  lax.fori_loop(0, NCH_RATE // 25, grp, 0)
  plsc.subcore_barrier()
  pltpu.sync_copy(hist_sh.at[pl.ds(s * (NI_PAD // NS), NI_PAD // NS)],
                  hist_out.at[pl.ds(c * NI_PAD + s * (NI_PAD // NS),
                                    NI_PAD // NS)])

  _zero_fill(pbuf, 8)

  def ssq_big(arr_hbm, k):
    def chunk_body(i, _):
      cidx = w + NW * i

      @pl.when(cidx < BIGC)
      def _():
        pltpu.sync_copy(arr_hbm.at[pl.ds(cidx * 2048, 2048)], vbuf)

        def red(f, aa):
          v = vbuf[pl.ds(f * L, L)]
          return aa + v * v

        acc = lax.fori_loop(0, 2048 // L, red, zeros16)
        pbuf[pl.ds(k * L, L)] = pbuf[pl.ds(k * L, L)] + acc

      return 0

    lax.fori_loop(0, 20, chunk_body, 0)

  def ssq_small(arr_hbm, k):
    @pl.when(w < NU // 400)
    def _():
      pltpu.sync_copy(arr_hbm.at[pl.ds(w * 400, 400)], vbuf.at[pl.ds(0, 400)])

      def red(f, aa):
        v = vbuf[pl.ds(f * L, L)]
        return aa + v * v

      acc = lax.fori_loop(0, 400 // L, red, zeros16)
      pbuf[pl.ds(k * L, L)] = acc

  ssq_small(bu_hbm, 0)
  ssq_big(pqu_hbm, 1)
  ssq_small(bi_hbm, 2)
  ssq_big(pqi_hbm, 3)
  ssq_big(ywi_hbm, 4)
  pltpu.sync_copy(pbuf, reg_out.at[pl.ds(w * 8 * L, 8 * L)])


@functools.partial(
    pl.kernel,
    out_type=jax.ShapeDtypeStruct((NI, D), jnp.float32),
    mesh=_MESH,
    scratch_types=[
        pltpu.VMEM((RK,), jnp.float32),
        pltpu.VMEM((RK,), jnp.float32),
        pltpu.VMEM((RK, D), jnp.float32),
    ],
)
def _k2a_scale(ywi_hbm, hist_hbm, syw_out, h0v, h1v, rows_v):
  c = lax.axis_index("c")
  s = lax.axis_index("s")
  w = c * NS + s
  iot = lax.iota(jnp.int32, L)

  def chunk(i, _):
    cidx = w + NW * i

    @pl.when(cidx < NCH_ROWS)
    def _():
      r0 = cidx * RK
      pltpu.sync_copy(hist_hbm.at[pl.ds(r0, RK)], h0v)
      pltpu.sync_copy(hist_hbm.at[pl.ds(NI_PAD + r0, RK)], h1v)
      pltpu.sync_copy(ywi_hbm.at[pl.ds(r0, RK)], rows_v)

      def grp(g, _2):
        h = h0v[pl.ds(g * L, L)] + h1v[pl.ds(g * L, L)]
        n = 1.0 / jnp.maximum(h, 1.0)

        def rloop(r2, _3):
          nr = _dyn_gather(n, jnp.broadcast_to(r2, (L,)))
          r = g * L + r2
          for f in range(D // L):
            sl = pl.ds(f * L, L)
            rows_v[r, sl] = rows_v[r, sl] * nr
          return 0

        lax.fori_loop(0, L, rloop, 0)
        return 0

      lax.fori_loop(0, RK // L, grp, 0)
      pltpu.sync_copy(rows_v, syw_out.at[pl.ds(r0, RK)])

    return 0

  lax.fori_loop(0, (NCH_ROWS + NW - 1) // NW, chunk, 0)


@functools.partial(
    pl.kernel,
    out_type=jax.ShapeDtypeStruct((NC, NU, D), jnp.float32),
    mesh=_MESH,
    scratch_types=[
        pltpu.VMEM((3, EK), jnp.int32),
        pltpu.VMEM((2, EK), jnp.int32),
        pltpu.VMEM((2, EK, D), jnp.float32),
        pltpu.VMEM((16, D), jnp.float32),
        pltpu.VMEM_SHARED((NU, D), jnp.float32),
        pltpu.SemaphoreType.DMA((3,)),
        pltpu.SemaphoreType.DMA((2,)),
        pltpu.SemaphoreType.DMA((2,)),
        pltpu.SemaphoreType.DMA((2,)),
    ],
)
def _k2b_segsum(rate_hbm, syw_hbm, acc_out, ivs, ivd, rows4, zbuf, acc_sh,
                issem, idsem, gsem, ssem):
  c = lax.axis_index("c")
  s = lax.axis_index("s")
  w = c * NS + s
  zeros16 = jnp.zeros((L,), jnp.float32)

  for r in range(16):
    for f in range(D // L):
      zbuf[r, pl.ds(f * L, L)] = zeros16

  n_grp = NU // 16

  def zloop(i, _):
    g16 = s + NS * i

    @pl.when(g16 < n_grp)
    def _():
      pltpu.sync_copy(zbuf, acc_sh.at[pl.ds(g16 * 16, 16)])

    return 0

  lax.fori_loop(0, (n_grp + NS - 1) // NS, zloop, 0)
  plsc.subcore_barrier()

  base_s = w * E_TILE
  base_d = E_RATE + w * E_TILE

  def isrc_start(j, k):
    pltpu.async_copy(rate_hbm.at[pl.ds(base_s + j * EK, EK)], ivs.at[k],
                     issem.at[k])

  def isrc_wait(j, k):
    pltpu.make_async_copy(rate_hbm.at[pl.ds(base_s + j * EK, EK)], ivs.at[k],
                          issem.at[k]).wait()

  def idst_start(j, k):
    pltpu.async_copy(rate_hbm.at[pl.ds(base_d + j * EK, EK)], ivd.at[k],
                     idsem.at[k])

  def idst_wait(j, k):
    pltpu.make_async_copy(rate_hbm.at[pl.ds(base_d + j * EK, EK)], ivd.at[k],
                          idsem.at[k]).wait()

  def gather_start(j, b):
    pltpu.async_copy(syw_hbm.at[ivd.at[lax.rem(j, 2)]], rows4.at[b],
                     gsem.at[b])

  def gather_wait(j, b):
    pltpu.make_async_copy(syw_hbm.at[ivd.at[lax.rem(j, 2)]], rows4.at[b],
                          gsem.at[b]).wait()

  def scat_start(j, b):
    pltpu.async_copy(rows4.at[b], acc_sh.at[ivs.at[lax.rem(j, 3)]],
                     ssem.at[b], add=True)

  def scat_wait(j, b):
    pltpu.make_async_copy(rows4.at[b], acc_sh.at[ivs.at[lax.rem(j, 3)]],
                          ssem.at[b]).wait()

  idst_start(0, 0)
  idst_start(1, 1)
  isrc_start(0, 0)
  isrc_start(1, 1)
  isrc_start(2, 2)
  idst_wait(0, 0)
  gather_start(0, 0)

  def eloop(j, _):
    b = lax.rem(j, 2)
    gather_wait(j, b)
    isrc_wait(j, lax.rem(j, 3))
    scat_start(j, b)

    @pl.when(j >= 1)
    def _():
      scat_wait(j - 1, 1 - b)

    @pl.when(j + 2 < NCH_RATE)
    def _():
      isrc_start(j + 2, lax.rem(j + 2, 3))

    @pl.when(j + 1 < NCH_RATE)
    def _():
      idst_wait(j + 1, 1 - b)
      gather_start(j + 1, 1 - b)

    @pl.when(j + 2 < NCH_RATE)
    def _():
      idst_start(j + 2, b)

    return 0

  lax.fori_loop(0, NCH_RATE, eloop, 0)
  scat_wait(NCH_RATE - 1, lax.rem(NCH_RATE - 1, 2))
  plsc.subcore_barrier()

  def wloop(i, _):
    g16 = s + NS * i

    @pl.when(g16 < n_grp)
    def _():
      r0 = g16 * 16
      pltpu.sync_copy(acc_sh.at[pl.ds(r0, 16)], acc_out.at[c, pl.ds(r0, 16)])

    return 0

  lax.fori_loop(0, (n_grp + NS - 1) // NS, wloop, 0)


@functools.partial(
    pl.kernel,
    out_type=jax.ShapeDtypeStruct((NU, D), jnp.float32),
    mesh=_MESH,
    scratch_types=[
        pltpu.VMEM((RK, D), jnp.float32),
        pltpu.VMEM((RK, D), jnp.float32),
        pltpu.VMEM((RK, D), jnp.float32),
        pltpu.VMEM((RK, D), jnp.float32),
    ],
)
def _k2c_hu(acc_hbm, pqu_hbm, hu_out, a0v, a1v, pv, hv):
  c = lax.axis_index("c")
  s = lax.axis_index("s")
  w = c * NS + s

  def chunk(i, _):
    cidx = w + NW * i

    @pl.when(cidx < NCH_ROWS)
    def _():
      r0 = cidx * RK
      pltpu.sync_copy(acc_hbm.at[0, pl.ds(r0, RK)], a0v)
      pltpu.sync_copy(acc_hbm.at[1, pl.ds(r0, RK)], a1v)
      pltpu.sync_copy(pqu_hbm.at[pl.ds(r0, RK)], pv)

      def rloop(r, _2):
        for f in range(D // L):
          sl = pl.ds(f * L, L)
          hv[r, sl] = a0v[r, sl] + a1v[r, sl] + pv[r, sl]
        return 0

      lax.fori_loop(0, RK, rloop, 0)
      pltpu.sync_copy(hv, hu_out.at[pl.ds(r0, RK)])

    return 0

  lax.fori_loop(0, (NCH_ROWS + NW - 1) // NW, chunk, 0)


@functools.partial(
    pl.kernel,
    out_type=(
        jax.ShapeDtypeStruct((EP_PAD,), jnp.float32),
        jax.ShapeDtypeStruct((EP_PAD,), jnp.float32),
    ),
    mesh=_MESH,
    scratch_types=[
        pltpu.VMEM((E0,), jnp.int32),
        pltpu.VMEM((E0,), jnp.int32),
        pltpu.VMEM((NB3, QK, D), jnp.float32),
        pltpu.VMEM((NB3, QK, D), jnp.float32),
        pltpu.VMEM((NB3, QK), jnp.float32),
        pltpu.VMEM((NB3, QK), jnp.float32),
        pltpu.VMEM((E0,), jnp.float32),
        pltpu.SemaphoreType.DMA((NB3,)),
        pltpu.SemaphoreType.DMA((NB3,)),
        pltpu.SemaphoreType.DMA((NB3,)),
        pltpu.SemaphoreType.DMA((NB3,)),
    ],
)
def _k3_score(pos_hbm, neg_hbm, hu_hbm, pqi_hbm, bu_hbm, bi_hbm,
              pos_out, neg_out, ivs, ivd, uv2, pv2, bu2, bi2, sv_all,
              usem, psem, busem, bisem):
  c = lax.axis_index("c")
  s = lax.axis_index("s")
  iot = lax.iota(jnp.int32, L)

  def run_set(edges_hbm, out_hbm, base, ne):
    nq = ne // QK
    pltpu.sync_copy(edges_hbm.at[pl.ds(base, ne)], ivs.at[pl.ds(0, ne)])
    pltpu.sync_copy(edges_hbm.at[pl.ds(EP_PAD + base, ne)],
                    ivd.at[pl.ds(0, ne)])

    def sidx(j):
      return ivs.at[pl.ds(j * QK, QK)]

    def didx(j):
      return ivd.at[pl.ds(j * QK, QK)]

    def gathers_start(j, b):
      pltpu.async_copy(hu_hbm.at[sidx(j)], uv2.at[b], usem.at[b])
      pltpu.async_copy(pqi_hbm.at[didx(j)], pv2.at[b], psem.at[b])
      pltpu.async_copy(bu_hbm.at[sidx(j)], bu2.at[b], busem.at[b])
      pltpu.async_copy(bi_hbm.at[didx(j)], bi2.at[b], bisem.at[b])

    def gathers_wait(j, b):
      pltpu.make_async_copy(hu_hbm.at[sidx(j)], uv2.at[b],
                            usem.at[b]).wait()
      pltpu.make_async_copy(pqi_hbm.at[didx(j)], pv2.at[b],
                            psem.at[b]).wait()
      pltpu.make_async_copy(bu_hbm.at[sidx(j)], bu2.at[b],
                            busem.at[b]).wait()
      pltpu.make_async_copy(bi_hbm.at[didx(j)], bi2.at[b],
                            bisem.at[b]).wait()

    for b in range(NB3 - 1):
      gathers_start(b, b)

    def chunk(j, _):
      b = lax.rem(j, NB3)
      gathers_wait(j, b)

      @pl.when(j + NB3 - 1 < nq)
      def _():
        gathers_start(j + NB3 - 1, lax.rem(j + NB3 - 1, NB3))

      for g in range(QK // L):

        def edge_body(r2, merged, g=g, b=b):
          r = g * L + r2
          acc = jnp.zeros((L,), jnp.float32)
          for f in range(D // L):
            sl = pl.ds(f * L, L)
            acc = acc + uv2[b, r, sl] * pv2[b, r, sl]
          for sh in (8, 4, 2, 1):
            acc = acc + _dyn_gather(acc, jnp.bitwise_xor(iot, sh))
          return jnp.where(iot == r2, acc, merged)

        dots = lax.fori_loop(0, L, edge_body, jnp.zeros((L,), jnp.float32))
        sc = (dots + bu2[b, pl.ds(g * L, L)] + bi2[b, pl.ds(g * L, L)] + GB)
        sv_all[pl.ds(j * QK + g * L, L)] = sc
      return 0

    lax.fori_loop(0, nq, chunk, 0)
    pltpu.sync_copy(sv_all.at[pl.ds(0, ne)], out_hbm.at[pl.ds(base, ne)])

  for edges_hbm, out_hbm in ((pos_hbm, pos_out), (neg_hbm, neg_out)):

    @pl.when(c == 0)
    def _(edges_hbm=edges_hbm, out_hbm=out_hbm):
      run_set(edges_hbm, out_hbm, s * (2 * EP_TILE), E0)

    @pl.when(c == 1)
    def _(edges_hbm=edges_hbm, out_hbm=out_hbm):
      run_set(edges_hbm, out_hbm, s * (2 * EP_TILE) + E0, E1)


def kernel(rate_edges, trust_edges, pos_edges, neg_edges,
           pq_user, pq_item, yw_user, yw_item, b_user, b_item):
  del trust_edges, yw_user
  pqu_flat = pq_user.reshape(-1)
  pqi_flat = pq_item.reshape(-1)
  ywi_flat = yw_item.reshape(-1)
  bu_flat = b_user.reshape(-1)
  bi_flat = b_item.reshape(-1)

  rate4 = rate_edges.reshape(2, NW, NCH_RATE, EK)
  hist, regp = _k1_hist_reg(rate4, pqu_flat, pqi_flat, ywi_flat,
                            bu_flat, bi_flat)
  syw = _k2a_scale(yw_item, hist)
  acc = _k2b_segsum(rate_edges.reshape(-1), syw)
  h_u = _k2c_hu(acc, pq_user)

  pos_f = jnp.pad(pos_edges, ((0, 0), (0, EP_PAD - E_PRED))).reshape(-1)
  neg_f = jnp.pad(neg_edges, ((0, 0), (0, EP_PAD - E_PRED))).reshape(-1)
  pos_s, neg_s = _k3_score(pos_f, neg_f, h_u, pq_item, bu_flat, bi_flat)
  pos_score = pos_s[:E_PRED, None]
  neg_score = neg_s[:E_PRED, None]

  ssq = regp.reshape(NW, 8, L)[:, :5, :].sum(axis=(0, 2))
  nb_u, np_u, nb_i, np_i, ny_i = [jnp.sqrt(ssq[k]) for k in range(5)]
  reg_loss = ((LAM * E_RATE / NU) * (nb_u + np_u) +
              (LAM * E_RATE / NI) * (nb_i + np_i + ny_i))
  return (pos_score, neg_score, reg_loss)
